# baseline (speedup 1.0000x reference)
import functools

import jax
import jax.numpy as jnp
from jax import lax
from jax.experimental import pallas as pl
from jax.experimental.pallas import tpu as pltpu
from jax.experimental.pallas import tpu_sc as plsc

_NC = 2
_NS = 16
_NW = _NC * _NS
_L = 16


@functools.lru_cache(maxsize=None)
def _build(n: int, tbl: int, chunk: int, rest: int):
    mesh = plsc.VectorSubcoreMesh(core_axis_name="c", subcore_axis_name="s")
    h1 = (rest // 2) // _L * _L
    h2 = rest - h1

    @functools.partial(
        pl.kernel,
        mesh=mesh,
        out_type=jax.ShapeDtypeStruct((n,), jnp.float32),
        scratch_types=[
            pltpu.VMEM((tbl,), jnp.float32),
            pltpu.VMEM((rest,), jnp.int32),
            pltpu.VMEM((rest,), jnp.float32),
            pltpu.VMEM((rest,), jnp.float32),
            pltpu.SemaphoreType.DMA,
            pltpu.SemaphoreType.DMA,
            pltpu.SemaphoreType.DMA,
        ],
        compiler_params=pltpu.CompilerParams(needs_layout_passes=False),
    )
    def sc_kernel(e_hbm, z_hbm, t_hbm, out_hbm, t_v, z_v, e_v, o_v,
                  sem_t, sem_z, sem_e):
        wid = lax.axis_index("s") * _NC + lax.axis_index("c")
        base = wid * chunk
        cp_t = pltpu.async_copy(t_hbm, t_v, sem_t)
        cp_z1 = pltpu.async_copy(z_hbm.at[pl.ds(base, h1)],
                                 z_v.at[pl.ds(0, h1)], sem_z)
        cp_e1 = pltpu.async_copy(e_hbm.at[pl.ds(base, h1)],
                                 e_v.at[pl.ds(0, h1)], sem_z)
        cp_z2 = pltpu.async_copy(z_hbm.at[pl.ds(base + h1, h2)],
                                 z_v.at[pl.ds(h1, h2)], sem_e)
        cp_e2 = pltpu.async_copy(e_hbm.at[pl.ds(base + h1, h2)],
                                 e_v.at[pl.ds(h1, h2)], sem_e)
        cp_t.wait()
        cp_z1.wait()
        cp_e1.wait()

        @pl.loop(0, h1 // _L, unroll=8)
        def step1(i):
            sl = pl.ds(i * _L, _L)
            scales = plsc.load_gather(t_v, [z_v[sl]])
            o_v[sl] = e_v[sl] * scales

        cp_z2.wait()
        cp_e2.wait()

        @pl.loop(0, h2 // _L, unroll=8)
        def step2(i):
            sl = pl.ds(h1 + i * _L, _L)
            scales = plsc.load_gather(t_v, [z_v[sl]])
            o_v[sl] = e_v[sl] * scales

        last = _NW - 1

        @pl.when(wid < last)
        def _():
            pltpu.sync_copy(o_v.at[pl.ds(0, chunk)],
                            out_hbm.at[pl.ds(base, chunk)])

        @pl.when(wid == last)
        def _():
            pltpu.sync_copy(o_v, out_hbm.at[pl.ds(base, rest)])

    return sc_kernel


@functools.lru_cache(maxsize=None)
def _build_padded(n_pad: int, tbl: int, chunk: int):
    mesh = plsc.VectorSubcoreMesh(core_axis_name="c", subcore_axis_name="s")

    @functools.partial(
        pl.kernel,
        mesh=mesh,
        out_type=jax.ShapeDtypeStruct((n_pad,), jnp.float32),
        scratch_types=[
            pltpu.VMEM((tbl,), jnp.float32),
            pltpu.VMEM((chunk,), jnp.int32),
            pltpu.VMEM((chunk,), jnp.float32),
            pltpu.VMEM((chunk,), jnp.float32),
        ],
        compiler_params=pltpu.CompilerParams(needs_layout_passes=False),
    )
    def sc_kernel(e_hbm, z_hbm, t_hbm, out_hbm, t_v, z_v, e_v, o_v):
        wid = lax.axis_index("s") * _NC + lax.axis_index("c")
        base = wid * chunk
        pltpu.sync_copy(t_hbm, t_v)
        pltpu.sync_copy(z_hbm.at[pl.ds(base, chunk)], z_v)
        pltpu.sync_copy(e_hbm.at[pl.ds(base, chunk)], e_v)

        @pl.loop(0, chunk // _L, unroll=8)
        def step(i):
            sl = pl.ds(i * _L, _L)
            scales = plsc.load_gather(t_v, [z_v[sl]])
            o_v[sl] = e_v[sl] * scales

        pltpu.sync_copy(o_v, out_hbm.at[pl.ds(base, chunk)])

    return sc_kernel


def kernel(local_energies, Z, per_element_scaling):
    e = jnp.squeeze(local_energies)
    n = e.shape[0]
    z = Z.astype(jnp.int32)
    t = per_element_scaling.astype(jnp.float32).reshape(-1)
    tbl = t.shape[0]

    if n % _L == 0 and (n // _NW) // _L > 0:
        chunk = (n // _NW) // _L * _L
        rest = n - (_NW - 1) * chunk
        return _build(n, tbl, chunk, rest)(e, z, t)

    chunk = -(-n // _NW)
    chunk = -(-chunk // _L) * _L
    n_pad = chunk * _NW
    e = jnp.pad(e, (0, n_pad - n))
    z = jnp.pad(z, (0, n_pad - n))
    out = _build_padded(n_pad, tbl, chunk)(e, z, t)
    return out[:n]


# speedup vs baseline: 1.0099x; 1.0099x over previous
import functools

import jax
import jax.numpy as jnp
from jax import lax
from jax.experimental import pallas as pl
from jax.experimental.pallas import tpu as pltpu
from jax.experimental.pallas import tpu_sc as plsc

_NC = 2
_NS = 16
_NW = _NC * _NS
_L = 16


@functools.lru_cache(maxsize=None)
def _build(n: int, tbl: int, chunk: int, rest: int):
    mesh = plsc.VectorSubcoreMesh(core_axis_name="c", subcore_axis_name="s")

    @functools.partial(
        pl.kernel,
        mesh=mesh,
        out_type=jax.ShapeDtypeStruct((n,), jnp.float32),
        scratch_types=[
            pltpu.VMEM((tbl,), jnp.float32),
            pltpu.VMEM((rest,), jnp.int32),
            pltpu.VMEM((rest,), jnp.float32),
            pltpu.VMEM((rest,), jnp.float32),
            pltpu.SemaphoreType.DMA,
            pltpu.SemaphoreType.DMA,
        ],
        compiler_params=pltpu.CompilerParams(needs_layout_passes=False),
    )
    def sc_kernel(e_hbm, z_hbm, t_hbm, out_hbm, t_v, z_v, e_v, o_v,
                  sem_z, sem_e):
        wid = lax.axis_index("s") * _NC + lax.axis_index("c")
        base = wid * chunk
        cp_z = pltpu.async_copy(z_hbm.at[pl.ds(base, rest)], z_v, sem_z)
        cp_e = pltpu.async_copy(e_hbm.at[pl.ds(base, rest)], e_v, sem_e)
        pltpu.sync_copy(t_hbm, t_v)
        cp_z.wait()
        cp_e.wait()

        @pl.loop(0, rest // _L, unroll=8)
        def step(i):
            sl = pl.ds(i * _L, _L)
            scales = plsc.load_gather(t_v, [z_v[sl]])
            o_v[sl] = e_v[sl] * scales

        pltpu.sync_copy(o_v, out_hbm.at[pl.ds(base, rest)])

    return sc_kernel


@functools.lru_cache(maxsize=None)
def _build_padded(n_pad: int, tbl: int, chunk: int):
    mesh = plsc.VectorSubcoreMesh(core_axis_name="c", subcore_axis_name="s")

    @functools.partial(
        pl.kernel,
        mesh=mesh,
        out_type=jax.ShapeDtypeStruct((n_pad,), jnp.float32),
        scratch_types=[
            pltpu.VMEM((tbl,), jnp.float32),
            pltpu.VMEM((chunk,), jnp.int32),
            pltpu.VMEM((chunk,), jnp.float32),
            pltpu.VMEM((chunk,), jnp.float32),
        ],
        compiler_params=pltpu.CompilerParams(needs_layout_passes=False),
    )
    def sc_kernel(e_hbm, z_hbm, t_hbm, out_hbm, t_v, z_v, e_v, o_v):
        wid = lax.axis_index("s") * _NC + lax.axis_index("c")
        base = wid * chunk
        pltpu.sync_copy(t_hbm, t_v)
        pltpu.sync_copy(z_hbm.at[pl.ds(base, chunk)], z_v)
        pltpu.sync_copy(e_hbm.at[pl.ds(base, chunk)], e_v)

        @pl.loop(0, chunk // _L, unroll=8)
        def step(i):
            sl = pl.ds(i * _L, _L)
            scales = plsc.load_gather(t_v, [z_v[sl]])
            o_v[sl] = e_v[sl] * scales

        pltpu.sync_copy(o_v, out_hbm.at[pl.ds(base, chunk)])

    return sc_kernel


def kernel(local_energies, Z, per_element_scaling):
    e = jnp.squeeze(local_energies)
    n = e.shape[0]
    z = Z.astype(jnp.int32)
    t = per_element_scaling.astype(jnp.float32).reshape(-1)
    tbl = t.shape[0]

    if n % _L == 0 and (n // _NW) // _L > 0:
        chunk = (n // _NW) // _L * _L
        rest = n - (_NW - 1) * chunk
        return _build(n, tbl, chunk, rest)(e, z, t)

    chunk = -(-n // _NW)
    chunk = -(-chunk // _L) * _L
    n_pad = chunk * _NW
    e = jnp.pad(e, (0, n_pad - n))
    z = jnp.pad(z, (0, n_pad - n))
    out = _build_padded(n_pad, tbl, chunk)(e, z, t)
    return out[:n]


# speedup vs baseline: 1.0812x; 1.0706x over previous
import functools

import jax
import jax.numpy as jnp
from jax import lax
from jax.experimental import pallas as pl
from jax.experimental.pallas import tpu as pltpu
from jax.experimental.pallas import tpu_sc as plsc

_NC = 2
_NS = 16
_NW = _NC * _NS
_L = 16


@functools.lru_cache(maxsize=None)
def _build(n: int, tbl: int, chunk: int, rest: int):
    mesh = plsc.VectorSubcoreMesh(core_axis_name="c", subcore_axis_name="s")

    @functools.partial(
        pl.kernel,
        mesh=mesh,
        out_type=jax.ShapeDtypeStruct((n,), jnp.float32),
        scratch_types=[
            pltpu.VMEM((tbl,), jnp.float32),
            pltpu.VMEM((rest,), jnp.int32),
            pltpu.VMEM((rest,), jnp.float32),
            pltpu.VMEM((rest,), jnp.float32),
            pltpu.SemaphoreType.DMA,
            pltpu.SemaphoreType.DMA,
        ],
        compiler_params=pltpu.CompilerParams(needs_layout_passes=False),
    )
    def sc_kernel(e_hbm, z_hbm, t_hbm, out_hbm, t_v, z_v, e_v, o_v,
                  sem_z, sem_e):
        wid = lax.axis_index("s") * _NC + lax.axis_index("c")
        base = wid * chunk
        cp_z = pltpu.async_copy(z_hbm.at[pl.ds(base, rest)], z_v, sem_z)
        cp_e = pltpu.async_copy(e_hbm.at[pl.ds(base, rest)], e_v, sem_e)
        pltpu.sync_copy(t_hbm, t_v)
        cp_z.wait()
        cp_e.wait()

        @plsc.parallel_loop(0, rest, _L, unroll=8)
        def step(i):
            sl = pl.ds(i, _L)
            scales = plsc.load_gather(t_v, [z_v[sl]])
            o_v[sl] = e_v[sl] * scales

        pltpu.sync_copy(o_v, out_hbm.at[pl.ds(base, rest)])

    return sc_kernel


@functools.lru_cache(maxsize=None)
def _build_padded(n_pad: int, tbl: int, chunk: int):
    mesh = plsc.VectorSubcoreMesh(core_axis_name="c", subcore_axis_name="s")

    @functools.partial(
        pl.kernel,
        mesh=mesh,
        out_type=jax.ShapeDtypeStruct((n_pad,), jnp.float32),
        scratch_types=[
            pltpu.VMEM((tbl,), jnp.float32),
            pltpu.VMEM((chunk,), jnp.int32),
            pltpu.VMEM((chunk,), jnp.float32),
            pltpu.VMEM((chunk,), jnp.float32),
        ],
        compiler_params=pltpu.CompilerParams(needs_layout_passes=False),
    )
    def sc_kernel(e_hbm, z_hbm, t_hbm, out_hbm, t_v, z_v, e_v, o_v):
        wid = lax.axis_index("s") * _NC + lax.axis_index("c")
        base = wid * chunk
        pltpu.sync_copy(t_hbm, t_v)
        pltpu.sync_copy(z_hbm.at[pl.ds(base, chunk)], z_v)
        pltpu.sync_copy(e_hbm.at[pl.ds(base, chunk)], e_v)

        @pl.loop(0, chunk // _L, unroll=8)
        def step(i):
            sl = pl.ds(i * _L, _L)
            scales = plsc.load_gather(t_v, [z_v[sl]])
            o_v[sl] = e_v[sl] * scales

        pltpu.sync_copy(o_v, out_hbm.at[pl.ds(base, chunk)])

    return sc_kernel


def kernel(local_energies, Z, per_element_scaling):
    e = jnp.squeeze(local_energies)
    n = e.shape[0]
    z = Z.astype(jnp.int32)
    t = per_element_scaling.astype(jnp.float32).reshape(-1)
    tbl = t.shape[0]

    if n % _L == 0 and (n // _NW) // _L > 0:
        chunk = (n // _NW) // _L * _L
        rest = n - (_NW - 1) * chunk
        return _build(n, tbl, chunk, rest)(e, z, t)

    chunk = -(-n // _NW)
    chunk = -(-chunk // _L) * _L
    n_pad = chunk * _NW
    e = jnp.pad(e, (0, n_pad - n))
    z = jnp.pad(z, (0, n_pad - n))
    out = _build_padded(n_pad, tbl, chunk)(e, z, t)
    return out[:n]
